# baseline (speedup 1.0000x reference)
import functools

import jax
import jax.numpy as jnp
from jax import lax
from jax.experimental import pallas as pl
from jax.experimental.pallas import tpu as pltpu
from jax.experimental.pallas import tpu_sc as plsc

B = 16
NPER = 625
E = 320000
DF = 256
DE = 4

R = NPER * NPER
WPG = R * DE
DUMMY_W = 1562504
SP_W = 1562512

NC, NS = 2, 16
G_PER_C = B // NC
EPW = E // NS
SEGE = 400
NSEG = EPW // SEGE
CE = 256
MAXCH = -(-SEGE // CE)
CHUNK0 = 2000
SENT = EPW
ZB = 512
ZSEG = 97664
ZSEG_L = SP_W - (NS - 1) * ZSEG
STRIPE = 97656
BWB = 2048


def _adj_body(src_hbm, dst_hbm, attr_hbm, zeros_hbm, out_hbm,
              acc, sdb, growv, ceid, idxc, attrseg, scb, bwb, zbuf, tsave,
              semg, sems, semw, semz):
    c = lax.axis_index("c")
    s = lax.axis_index("s")
    ebase = s * EPW
    iot = lax.iota(jnp.int32, 16)
    P = iot // 4
    OFFS = iot % 4

    pltpu.sync_copy(zeros_hbm, zbuf)

    def src_pass(ci, carry):
        off = ci * CHUNK0
        pltpu.sync_copy(src_hbm.at[pl.ds(ebase + off, CHUNK0)], sdb)

        def vec_body(i, carry2):
            sv = sdb[pl.ds(i * 16, 16)]
            growv[pl.ds(off + i * 16, 16)] = (
                sv * (NPER * DE) - (sv // NPER) * (NPER * DE))
            return carry2

        lax.fori_loop(0, CHUNK0 // 16, vec_body, 0)
        return carry

    lax.fori_loop(0, EPW // CHUNK0, src_pass, 0)

    def dst_pass(ci, carry):
        off = ci * CHUNK0
        pltpu.sync_copy(dst_hbm.at[pl.ds(ebase + off, CHUNK0)], sdb)

        def vec_body(i, carry2):
            dv = sdb[pl.ds(i * 16, 16)]
            w = growv[pl.ds(off + i * 16, 16)]
            growv[pl.ds(off + i * 16, 16)] = w + dv * DE
            return carry2

        lax.fori_loop(0, CHUNK0 // 16, vec_body, 0)
        return carry

    lax.fori_loop(0, EPW // CHUNK0, dst_pass, 0)

    def zloop(k, carry):
        ds_ = [pltpu.async_copy(
                   zbuf, acc.at[pl.ds(s * ZSEG + (k * 10 + j) * ZB, ZB)],
                   semz) for j in range(10)]
        for d in ds_:
            d.wait()
        return carry

    lax.fori_loop(0, 19, zloop, 0)

    @pl.when(s < NS - 1)
    def _():
        pltpu.sync_copy(zbuf.at[pl.ds(0, ZSEG - 190 * ZB)],
                        acc.at[pl.ds(s * ZSEG + 190 * ZB, ZSEG - 190 * ZB)])

    @pl.when(s == NS - 1)
    def _():
        pltpu.sync_copy(zbuf.at[pl.ds(0, ZSEG_L - 190 * ZB)],
                        acc.at[pl.ds(s * ZSEG + 190 * ZB, ZSEG_L - 190 * ZB)])

    plsc.subcore_barrier()

    for t in range(G_PER_C):
        g = c * G_PER_C + t
        wg = g * WPG
        sh = 4 * (t % 2)
        cwbase = c * (G_PER_C * WPG)
        toff = t * WPG + sh
        wbias = wg - sh

        if t % 2:
            @pl.when(s == 0)
            def _():
                pltpu.sync_copy(tsave, acc.at[pl.ds(0, 8)])

            plsc.subcore_barrier()

        growv[pl.ds(EPW, 16)] = jnp.full((16,), DUMMY_W, jnp.int32) + wbias

        def compact_seg(seg, wbias=wbias, sh=sh):
            def cp_body(i, off):
                e0 = seg * SEGE + i * 16
                w = growv[pl.ds(e0, 16)]
                l = w - wbias
                ok = (l >= sh) & (l < WPG + sh)
                plsc.store_compressed(ceid.at[pl.ds(off, 16)],
                                      e0 + iot, mask=ok)
                cnt = plsc.all_reduce_population_count(ok)
                return off + cnt[0]

            nc = lax.fori_loop(0, SEGE // 16, cp_body, 0)

            def pad_body(k, carry, nc=nc):
                ceid[pl.ds(nc + k * 16, 16)] = SENT + iot
                return carry

            lax.fori_loop(0, CE // 16, pad_body, 0)
            return nc

        def build_chunk(k, segbase, with_attr, wbias=wbias):
            def build(m, carry2):
                e4 = plsc.load_gather(ceid, [k * CE + m * 4 + P])
                w16 = plsc.load_gather(growv, [e4])
                idxc[m // 8, 0, pl.ds((m % 8) * 16, 16)] = (
                    w16 - wbias + OFFS)
                if with_attr:
                    aw = jnp.minimum((e4 - segbase) * DE,
                                     SEGE * DE - DE) + OFFS
                    scb[pl.ds(m * 16, 16)] = plsc.load_gather(attrseg, [aw])
                return carry2

            lax.fori_loop(0, CE // 4, build, 0)

        def seg_sc(seg, carry):
            segbase = seg * SEGE
            ad = pltpu.async_copy(
                attr_hbm.at[pl.ds((ebase + segbase) * DE, SEGE * DE)],
                attrseg, semg)
            nc = compact_seg(seg)
            ad.wait()

            def chunk_sc(k, carry2, nc=nc, segbase=segbase):
                @pl.when(k * CE < nc)
                def _():
                    build_chunk(k, segbase, True)

                    ss = [pltpu.async_copy(scb.at[pl.ds(r * 128, 128)],
                                           acc.at[idxc.at[r, 0]],
                                           sems, add=True)
                          for r in range(CE // 32)]
                    for d in ss:
                        d.wait()

                return carry2

            lax.fori_loop(0, MAXCH, chunk_sc, 0)
            return carry

        lax.fori_loop(0, NSEG, seg_sc, 0)
        plsc.subcore_barrier()

        if t % 2 == 0:
            @pl.when(s == 0)
            def _():
                pltpu.sync_copy(acc.at[pl.ds(WPG - 4, 8)], tsave)

        wprev = None
        for k in range(STRIPE // BWB):
            half = bwb.at[pl.ds((k % 2) * BWB, BWB)]
            pltpu.sync_copy(acc.at[pl.ds(2 * sh + s * STRIPE + k * BWB,
                                         BWB)], half)
            if wprev is not None:
                wprev.wait()
            wprev = pltpu.async_copy(
                half,
                out_hbm.at[pl.ds(cwbase + toff + s * STRIPE + k * BWB,
                                 BWB)], semw)
        wt = STRIPE % BWB
        kt = STRIPE // BWB
        half = bwb.at[pl.ds((kt % 2) * BWB, wt)]
        pltpu.sync_copy(
            acc.at[pl.ds(2 * sh + s * STRIPE + (STRIPE - wt), wt)], half)
        wprev.wait()
        pltpu.sync_copy(
            half,
            out_hbm.at[pl.ds(cwbase + toff + s * STRIPE + (STRIPE - wt), wt)])

        if t % 2:
            @pl.when(s == 0)
            def _():
                pltpu.sync_copy(acc.at[pl.ds(0, 8)], tsave)
                pltpu.sync_copy(tsave,
                                out_hbm.at[pl.ds(cwbase + t * WPG - 4, 8)])

        plsc.subcore_barrier()

        if t < G_PER_C - 1:
            def seg_rz(seg, carry):
                nc = compact_seg(seg)

                def chunk_rz(k, carry2, nc=nc, segbase=seg * SEGE):
                    @pl.when(k * CE < nc)
                    def _():
                        build_chunk(k, segbase, False)

                        zs = [pltpu.async_copy(zbuf.at[pl.ds(0, 128)],
                                               acc.at[idxc.at[r, 0]],
                                               semz)
                              for r in range(CE // 32)]
                        for d in zs:
                            d.wait()

                    return carry2

                lax.fori_loop(0, MAXCH, chunk_rz, 0)
                return carry

            lax.fori_loop(0, NSEG, seg_rz, 0)

            if t % 2:
                @pl.when(s == 0)
                def _():
                    pltpu.sync_copy(zbuf.at[pl.ds(0, 8)], acc.at[pl.ds(0, 8)])

            plsc.subcore_barrier()


_adj_call = functools.partial(
    pl.kernel,
    out_type=jax.ShapeDtypeStruct((B * WPG,), jnp.float32),
    mesh=plsc.VectorSubcoreMesh(core_axis_name="c", subcore_axis_name="s",
                                num_cores=NC, num_subcores=NS),
    compiler_params=pltpu.CompilerParams(needs_layout_passes=False),
    scratch_types=[
        pltpu.VMEM_SHARED((SP_W,), jnp.float32),
        pltpu.VMEM((CHUNK0,), jnp.int32),
        pltpu.VMEM((EPW + 16,), jnp.int32),
        pltpu.VMEM((SEGE + CE + 48,), jnp.int32),
        pltpu.VMEM((CE // 32, 1, 128), jnp.int32),
        pltpu.VMEM((SEGE * DE,), jnp.float32),
        pltpu.VMEM((CE * 4,), jnp.float32),
        pltpu.VMEM((2 * BWB,), jnp.float32),
        pltpu.VMEM((ZB,), jnp.float32),
        pltpu.VMEM((8,), jnp.float32),
        pltpu.SemaphoreType.DMA,
        pltpu.SemaphoreType.DMA,
        pltpu.SemaphoreType.DMA,
        pltpu.SemaphoreType.DMA,
    ],
)(_adj_body)


def kernel(x, edge_index, edge_attr, batch):
    src = edge_index[0]
    dst = edge_index[1]
    zeros = jnp.zeros((ZB,), jnp.float32)
    attr_pad = jnp.concatenate(
        [edge_attr.reshape(-1), jnp.zeros((64,), jnp.float32)])
    adj = _adj_call(src, dst, attr_pad, zeros)
    return adj.reshape(B, NPER, NPER, DE), x.reshape(B, NPER, DF)


# speedup vs baseline: 1.4359x; 1.4359x over previous
import functools

import jax
import jax.numpy as jnp
from jax import lax
from jax.experimental import pallas as pl
from jax.experimental.pallas import tpu as pltpu
from jax.experimental.pallas import tpu_sc as plsc

B = 16
NPER = 625
E = 320000
DF = 256
DE = 4

R = NPER * NPER
WPG = R * DE
DUMMY_W = 1562504
SP_W = 1562512

NC, NS = 2, 16
G_PER_C = B // NC
EPW = E // NS
SEGE = 2000
NSEG = EPW // SEGE
CE = 256
MAXCH = -(-SEGE // CE)
SENT = EPW
ZB = 512
ZSEG = 97664
ZSEG_L = SP_W - (NS - 1) * ZSEG
STRIPE = 97656
BWB = 2048
CL = 2560


def _adj_body(src_hbm, dst_hbm, attr_hbm, zeros_hbm, out_hbm, lists_hbm,
              counts_hbm, acc, sdb, growv, ceid, idxc, aidxc, scb, bwb,
              zbuf, tsave, cbuf, semg, sems, semw, semz):
    c = lax.axis_index("c")
    s = lax.axis_index("s")
    ebase = s * EPW
    iot = lax.iota(jnp.int32, 16)
    P = iot // 4
    OFFS = iot % 4

    pltpu.sync_copy(zeros_hbm, zbuf)

    def src_pass(ci, carry):
        off = ci * SEGE
        pltpu.sync_copy(src_hbm.at[pl.ds(ebase + off, SEGE)], sdb)

        def vec_body(i, carry2):
            sv = sdb[pl.ds(i * 16, 16)]
            growv[pl.ds(off + i * 16, 16)] = (
                sv * (NPER * DE) - (sv // NPER) * (NPER * DE))
            return carry2

        lax.fori_loop(0, SEGE // 16, vec_body, 0)
        return carry

    lax.fori_loop(0, NSEG, src_pass, 0)

    def dst_pass(ci, carry):
        off = ci * SEGE
        pltpu.sync_copy(dst_hbm.at[pl.ds(ebase + off, SEGE)], sdb)

        def vec_body(i, carry2):
            dv = sdb[pl.ds(i * 16, 16)]
            w = growv[pl.ds(off + i * 16, 16)]
            growv[pl.ds(off + i * 16, 16)] = w + dv * DE
            return carry2

        lax.fori_loop(0, SEGE // 16, vec_body, 0)
        return carry

    lax.fori_loop(0, NSEG, dst_pass, 0)

    def zloop(k, carry):
        ds_ = [pltpu.async_copy(
                   zbuf, acc.at[pl.ds(s * ZSEG + (k * 10 + j) * ZB, ZB)],
                   semz) for j in range(10)]
        for d in ds_:
            d.wait()
        return carry

    lax.fori_loop(0, 19, zloop, 0)

    @pl.when(s < NS - 1)
    def _():
        pltpu.sync_copy(zbuf.at[pl.ds(0, ZSEG - 190 * ZB)],
                        acc.at[pl.ds(s * ZSEG + 190 * ZB, ZSEG - 190 * ZB)])

    @pl.when(s == NS - 1)
    def _():
        pltpu.sync_copy(zbuf.at[pl.ds(0, ZSEG_L - 190 * ZB)],
                        acc.at[pl.ds(s * ZSEG + 190 * ZB, ZSEG_L - 190 * ZB)])

    plsc.subcore_barrier()

    for t in range(G_PER_C):
        g = c * G_PER_C + t
        wg = g * WPG
        sh = 4 * (t % 2)
        cwbase = c * (G_PER_C * WPG)
        toff = t * WPG + sh
        wbias = wg - sh

        if t % 2:
            @pl.when(s == 0)
            def _():
                pltpu.sync_copy(tsave, acc.at[pl.ds(0, 8)])

            plsc.subcore_barrier()

        growv[pl.ds(EPW, 16)] = jnp.full((16,), DUMMY_W, jnp.int32) + wbias

        def compact_seg(seg, wbias=wbias, sh=sh):
            def cp_body(i, off):
                e0 = seg * SEGE + i * 16
                w = growv[pl.ds(e0, 16)]
                l = w - wbias
                ok = (l >= sh) & (l < WPG + sh)
                plsc.store_compressed(ceid.at[pl.ds(off, 16)],
                                      e0 + iot, mask=ok)
                cnt = plsc.all_reduce_population_count(ok)
                return off + cnt[0]

            nc = lax.fori_loop(0, SEGE // 16, cp_body, 0)

            def pad_body(k, carry, nc=nc):
                ceid[pl.ds(nc + k * 16, 16)] = SENT + iot
                return carry

            lax.fori_loop(0, CE // 16 * 2, pad_body, 0)
            return nc

        def build_chunk(k, with_attr, wbias=wbias):
            def build(m, carry2):
                e4 = plsc.load_gather(ceid, [k * CE + m * 4 + P])
                w16 = plsc.load_gather(growv, [e4])
                idxc[m // 8, 0, pl.ds((m % 8) * 16, 16)] = (
                    w16 - wbias + OFFS)
                if with_attr:
                    aidxc[m // 8, 0, pl.ds((m % 8) * 16, 16)] = (
                        (e4 + ebase) * DE + OFFS)
                return carry2

            lax.fori_loop(0, CE // 4, build, 0)

        lbase = ((c * NS + s) * G_PER_C + t) * (NSEG * CL)

        def seg_sc(seg, cntv, lbase=lbase):
            nc = compact_seg(seg)
            cntv = jnp.where(iot == seg, nc, cntv)
            sv_ = pltpu.async_copy(ceid, lists_hbm.at[pl.ds(lbase + seg * CL,
                                                            CL)], semw)

            def chunk_sc(k, carry2, nc=nc):
                @pl.when(k * CE < nc)
                def _():
                    build_chunk(k, True)

                    gs = [pltpu.async_copy(attr_hbm.at[aidxc.at[r, 0]],
                                           scb.at[pl.ds(r * 128, 128)],
                                           semg)
                          for r in range(CE // 32)]
                    for d in gs:
                        d.wait()
                    ss = [pltpu.async_copy(scb.at[pl.ds(r * 128, 128)],
                                           acc.at[idxc.at[r, 0]],
                                           sems, add=True)
                          for r in range(CE // 32)]
                    for d in ss:
                        d.wait()

                return carry2

            lax.fori_loop(0, MAXCH, chunk_sc, 0)
            sv_.wait()
            return cntv

        cntv = lax.fori_loop(0, NSEG, seg_sc, jnp.zeros((16,), jnp.int32))
        cbuf[pl.ds(0, 16)] = cntv
        pltpu.sync_copy(cbuf, counts_hbm.at[pl.ds(
            ((c * NS + s) * G_PER_C + t) * 16, 16)])
        plsc.subcore_barrier()

        if t % 2 == 0:
            @pl.when(s == 0)
            def _():
                pltpu.sync_copy(acc.at[pl.ds(WPG - 4, 8)], tsave)

        wprev = None
        for k in range(STRIPE // BWB):
            half = bwb.at[pl.ds((k % 2) * BWB, BWB)]
            pltpu.sync_copy(acc.at[pl.ds(2 * sh + s * STRIPE + k * BWB,
                                         BWB)], half)
            if wprev is not None:
                wprev.wait()
            wprev = pltpu.async_copy(
                half,
                out_hbm.at[pl.ds(cwbase + toff + s * STRIPE + k * BWB,
                                 BWB)], semw)
        wt = STRIPE % BWB
        kt = STRIPE // BWB
        half = bwb.at[pl.ds((kt % 2) * BWB, wt)]
        pltpu.sync_copy(
            acc.at[pl.ds(2 * sh + s * STRIPE + (STRIPE - wt), wt)], half)
        wprev.wait()
        pltpu.sync_copy(
            half,
            out_hbm.at[pl.ds(cwbase + toff + s * STRIPE + (STRIPE - wt), wt)])

        if t % 2:
            @pl.when(s == 0)
            def _():
                pltpu.sync_copy(acc.at[pl.ds(0, 8)], tsave)
                pltpu.sync_copy(tsave,
                                out_hbm.at[pl.ds(cwbase + t * WPG - 4, 8)])

        plsc.subcore_barrier()

        if t < G_PER_C - 1:
            pltpu.sync_copy(counts_hbm.at[pl.ds(
                ((c * NS + s) * G_PER_C + t) * 16, 16)], cbuf)

            def seg_rz(seg, carry, lbase=lbase):
                pltpu.sync_copy(lists_hbm.at[pl.ds(lbase + seg * CL, CL)],
                                ceid)
                cv = cbuf[pl.ds(0, 16)]
                nc = jnp.max(jnp.where(iot == seg, cv, 0))

                def chunk_rz(k, carry2, nc=nc):
                    @pl.when(k * CE < nc)
                    def _():
                        build_chunk(k, False)

                        zs = [pltpu.async_copy(zbuf.at[pl.ds(0, 128)],
                                               acc.at[idxc.at[r, 0]],
                                               semz)
                              for r in range(CE // 32)]
                        for d in zs:
                            d.wait()

                    return carry2

                lax.fori_loop(0, MAXCH, chunk_rz, 0)
                return carry

            lax.fori_loop(0, NSEG, seg_rz, 0)

            if t % 2:
                @pl.when(s == 0)
                def _():
                    pltpu.sync_copy(zbuf.at[pl.ds(0, 8)], acc.at[pl.ds(0, 8)])

            plsc.subcore_barrier()


_adj_call = functools.partial(
    pl.kernel,
    out_type=(jax.ShapeDtypeStruct((B * WPG,), jnp.float32),
              jax.ShapeDtypeStruct((NC * NS * G_PER_C * NSEG * CL,),
                                   jnp.int32),
              jax.ShapeDtypeStruct((NC * NS * G_PER_C * 16,), jnp.int32)),
    mesh=plsc.VectorSubcoreMesh(core_axis_name="c", subcore_axis_name="s",
                                num_cores=NC, num_subcores=NS),
    compiler_params=pltpu.CompilerParams(needs_layout_passes=False),
    scratch_types=[
        pltpu.VMEM_SHARED((SP_W,), jnp.float32),
        pltpu.VMEM((SEGE,), jnp.int32),
        pltpu.VMEM((EPW + 16,), jnp.int32),
        pltpu.VMEM((SEGE + 2 * CE + 48,), jnp.int32),
        pltpu.VMEM((CE // 32, 1, 128), jnp.int32),
        pltpu.VMEM((CE // 32, 1, 128), jnp.int32),
        pltpu.VMEM((CE * 4,), jnp.float32),
        pltpu.VMEM((2 * BWB,), jnp.float32),
        pltpu.VMEM((ZB,), jnp.float32),
        pltpu.VMEM((8,), jnp.float32),
        pltpu.VMEM((16,), jnp.int32),
        pltpu.SemaphoreType.DMA,
        pltpu.SemaphoreType.DMA,
        pltpu.SemaphoreType.DMA,
        pltpu.SemaphoreType.DMA,
    ],
)(_adj_body)


def kernel(x, edge_index, edge_attr, batch):
    src = edge_index[0]
    dst = edge_index[1]
    zeros = jnp.zeros((ZB,), jnp.float32)
    attr_pad = jnp.concatenate(
        [edge_attr.reshape(-1), jnp.zeros((64,), jnp.float32)])
    adj, _, _ = _adj_call(src, dst, attr_pad, zeros)
    return adj.reshape(B, NPER, NPER, DE), x.reshape(B, NPER, DF)


# speedup vs baseline: 1.4419x; 1.0042x over previous
import functools

import jax
import jax.numpy as jnp
from jax import lax
from jax.experimental import pallas as pl
from jax.experimental.pallas import tpu as pltpu
from jax.experimental.pallas import tpu_sc as plsc

B = 16
NPER = 625
E = 320000
DF = 256
DE = 4

R = NPER * NPER
WPG = R * DE
DUMMY_W = 1562504
SP_W = 1562512

NC, NS = 2, 16
G_PER_C = B // NC
EPW = E // NS
SEGE = 2000
NSEG = EPW // SEGE
CE = 256
MAXCH = -(-SEGE // CE)
SENT = EPW
ZB = 512
ZSEG = 97664
ZSEG_L = SP_W - (NS - 1) * ZSEG
STRIPE = 97656
BWB = 2048
CL = 2560


def _adj_body(src_hbm, dst_hbm, attr_hbm, zeros_hbm, out_hbm, lists_hbm,
              counts_hbm, acc, sdb, growv, ceid, idxc, aidxc, scb, bwb,
              zbuf, tsave, cbuf, semg, sems, semw, semz):
    c = lax.axis_index("c")
    s = lax.axis_index("s")
    ebase = s * EPW
    iot = lax.iota(jnp.int32, 16)
    P = iot // 4
    OFFS = iot % 4

    pltpu.sync_copy(zeros_hbm, zbuf)

    def src_pass(ci, carry):
        off = ci * SEGE
        pltpu.sync_copy(src_hbm.at[pl.ds(ebase + off, SEGE)], sdb)

        def vec_body(i, carry2):
            sv = sdb[pl.ds(i * 16, 16)]
            growv[pl.ds(off + i * 16, 16)] = (
                sv * (NPER * DE) - (sv // NPER) * (NPER * DE))
            return carry2

        lax.fori_loop(0, SEGE // 16, vec_body, 0)
        return carry

    lax.fori_loop(0, NSEG, src_pass, 0)

    def dst_pass(ci, carry):
        off = ci * SEGE
        pltpu.sync_copy(dst_hbm.at[pl.ds(ebase + off, SEGE)], sdb)

        def vec_body(i, carry2):
            dv = sdb[pl.ds(i * 16, 16)]
            w = growv[pl.ds(off + i * 16, 16)]
            growv[pl.ds(off + i * 16, 16)] = w + dv * DE
            return carry2

        lax.fori_loop(0, SEGE // 16, vec_body, 0)
        return carry

    lax.fori_loop(0, NSEG, dst_pass, 0)

    def zloop(k, carry):
        ds_ = [pltpu.async_copy(
                   zbuf, acc.at[pl.ds(s * ZSEG + (k * 10 + j) * ZB, ZB)],
                   semz) for j in range(10)]
        for d in ds_:
            d.wait()
        return carry

    lax.fori_loop(0, 19, zloop, 0)

    @pl.when(s < NS - 1)
    def _():
        pltpu.sync_copy(zbuf.at[pl.ds(0, ZSEG - 190 * ZB)],
                        acc.at[pl.ds(s * ZSEG + 190 * ZB, ZSEG - 190 * ZB)])

    @pl.when(s == NS - 1)
    def _():
        pltpu.sync_copy(zbuf.at[pl.ds(0, ZSEG_L - 190 * ZB)],
                        acc.at[pl.ds(s * ZSEG + 190 * ZB, ZSEG_L - 190 * ZB)])

    plsc.subcore_barrier()

    for t in range(G_PER_C):
        g = c * G_PER_C + t
        wg = g * WPG
        sh = 4 * (t % 2)
        cwbase = c * (G_PER_C * WPG)
        toff = t * WPG + sh
        wbias = wg - sh

        if t % 2:
            @pl.when(s == 0)
            def _():
                pltpu.sync_copy(tsave, acc.at[pl.ds(0, 8)])

            plsc.subcore_barrier()

        growv[pl.ds(EPW, 16)] = jnp.full((16,), DUMMY_W, jnp.int32) + wbias

        def compact_seg(seg, wbias=wbias, sh=sh):
            def cp_body(i, offv):
                e0 = seg * SEGE + i * 16
                w = growv[pl.ds(e0, 16)]
                l = w - wbias
                ok = (l >= sh) & (l < WPG + sh)
                ci = plsc.cumsum(jnp.where(ok, 1, 0))
                plsc.store_scatter(ceid, [offv + ci - 1], e0 + iot, mask=ok)
                return offv + plsc.all_reduce_population_count(ok)

            offv = lax.fori_loop(0, SEGE // 16, cp_body,
                                 jnp.zeros((16,), jnp.int32))
            nc = offv[0]

            def pad_body(k, carry, nc=nc):
                ceid[pl.ds(nc + k * 16, 16)] = SENT + iot
                return carry

            lax.fori_loop(0, CE // 16 * 2, pad_body, 0)
            return nc

        def build_chunk(k, with_attr, wbias=wbias):
            def build(m, carry2):
                e4 = plsc.load_gather(ceid, [k * CE + m * 4 + P])
                w16 = plsc.load_gather(growv, [e4])
                idxc[m // 8, 0, pl.ds((m % 8) * 16, 16)] = (
                    w16 - wbias + OFFS)
                if with_attr:
                    aidxc[m // 8, 0, pl.ds((m % 8) * 16, 16)] = (
                        (e4 + ebase) * DE + OFFS)
                return carry2

            lax.fori_loop(0, CE // 4, build, 0)

        lbase = ((c * NS + s) * G_PER_C + t) * (NSEG * CL)

        def seg_sc(seg, cntv, lbase=lbase):
            nc = compact_seg(seg)
            cntv = jnp.where(iot == seg, nc, cntv)
            sv_ = pltpu.async_copy(ceid, lists_hbm.at[pl.ds(lbase + seg * CL,
                                                            CL)], semw)

            def chunk_sc(k, carry2, nc=nc):
                @pl.when(k * CE < nc)
                def _():
                    build_chunk(k, True)

                    gs = [pltpu.async_copy(attr_hbm.at[aidxc.at[r, 0]],
                                           scb.at[pl.ds(r * 128, 128)],
                                           semg)
                          for r in range(CE // 32)]
                    for d in gs:
                        d.wait()
                    ss = [pltpu.async_copy(scb.at[pl.ds(r * 128, 128)],
                                           acc.at[idxc.at[r, 0]],
                                           sems, add=True)
                          for r in range(CE // 32)]
                    for d in ss:
                        d.wait()

                return carry2

            lax.fori_loop(0, MAXCH, chunk_sc, 0)
            sv_.wait()
            return cntv

        cntv = lax.fori_loop(0, NSEG, seg_sc, jnp.zeros((16,), jnp.int32))
        cbuf[pl.ds(0, 16)] = cntv
        pltpu.sync_copy(cbuf, counts_hbm.at[pl.ds(
            ((c * NS + s) * G_PER_C + t) * 16, 16)])
        plsc.subcore_barrier()

        if t % 2 == 0:
            @pl.when(s == 0)
            def _():
                pltpu.sync_copy(acc.at[pl.ds(WPG - 4, 8)], tsave)

        wprev = None
        for k in range(STRIPE // BWB):
            half = bwb.at[pl.ds((k % 2) * BWB, BWB)]
            pltpu.sync_copy(acc.at[pl.ds(2 * sh + s * STRIPE + k * BWB,
                                         BWB)], half)
            if wprev is not None:
                wprev.wait()
            wprev = pltpu.async_copy(
                half,
                out_hbm.at[pl.ds(cwbase + toff + s * STRIPE + k * BWB,
                                 BWB)], semw)
        wt = STRIPE % BWB
        kt = STRIPE // BWB
        half = bwb.at[pl.ds((kt % 2) * BWB, wt)]
        pltpu.sync_copy(
            acc.at[pl.ds(2 * sh + s * STRIPE + (STRIPE - wt), wt)], half)
        wprev.wait()
        pltpu.sync_copy(
            half,
            out_hbm.at[pl.ds(cwbase + toff + s * STRIPE + (STRIPE - wt), wt)])

        if t % 2:
            @pl.when(s == 0)
            def _():
                pltpu.sync_copy(acc.at[pl.ds(0, 8)], tsave)
                pltpu.sync_copy(tsave,
                                out_hbm.at[pl.ds(cwbase + t * WPG - 4, 8)])

        plsc.subcore_barrier()

        if t < G_PER_C - 1:
            pltpu.sync_copy(counts_hbm.at[pl.ds(
                ((c * NS + s) * G_PER_C + t) * 16, 16)], cbuf)

            def seg_rz(seg, carry, lbase=lbase):
                pltpu.sync_copy(lists_hbm.at[pl.ds(lbase + seg * CL, CL)],
                                ceid)
                cv = cbuf[pl.ds(0, 16)]
                nc = jnp.max(jnp.where(iot == seg, cv, 0))

                def chunk_rz(k, carry2, nc=nc):
                    @pl.when(k * CE < nc)
                    def _():
                        build_chunk(k, False)

                        zs = [pltpu.async_copy(zbuf.at[pl.ds(0, 128)],
                                               acc.at[idxc.at[r, 0]],
                                               semz)
                              for r in range(CE // 32)]
                        for d in zs:
                            d.wait()

                    return carry2

                lax.fori_loop(0, MAXCH, chunk_rz, 0)
                return carry

            lax.fori_loop(0, NSEG, seg_rz, 0)

            if t % 2:
                @pl.when(s == 0)
                def _():
                    pltpu.sync_copy(zbuf.at[pl.ds(0, 8)], acc.at[pl.ds(0, 8)])

            plsc.subcore_barrier()


_adj_call = functools.partial(
    pl.kernel,
    out_type=(jax.ShapeDtypeStruct((B * WPG,), jnp.float32),
              jax.ShapeDtypeStruct((NC * NS * G_PER_C * NSEG * CL,),
                                   jnp.int32),
              jax.ShapeDtypeStruct((NC * NS * G_PER_C * 16,), jnp.int32)),
    mesh=plsc.VectorSubcoreMesh(core_axis_name="c", subcore_axis_name="s",
                                num_cores=NC, num_subcores=NS),
    compiler_params=pltpu.CompilerParams(needs_layout_passes=False),
    scratch_types=[
        pltpu.VMEM_SHARED((SP_W,), jnp.float32),
        pltpu.VMEM((SEGE,), jnp.int32),
        pltpu.VMEM((EPW + 16,), jnp.int32),
        pltpu.VMEM((SEGE + 2 * CE + 48,), jnp.int32),
        pltpu.VMEM((CE // 32, 1, 128), jnp.int32),
        pltpu.VMEM((CE // 32, 1, 128), jnp.int32),
        pltpu.VMEM((CE * 4,), jnp.float32),
        pltpu.VMEM((2 * BWB,), jnp.float32),
        pltpu.VMEM((ZB,), jnp.float32),
        pltpu.VMEM((8,), jnp.float32),
        pltpu.VMEM((16,), jnp.int32),
        pltpu.SemaphoreType.DMA,
        pltpu.SemaphoreType.DMA,
        pltpu.SemaphoreType.DMA,
        pltpu.SemaphoreType.DMA,
    ],
)(_adj_body)


def kernel(x, edge_index, edge_attr, batch):
    src = edge_index[0]
    dst = edge_index[1]
    zeros = jnp.zeros((ZB,), jnp.float32)
    attr_pad = jnp.concatenate(
        [edge_attr.reshape(-1), jnp.zeros((64,), jnp.float32)])
    adj, _, _ = _adj_call(src, dst, attr_pad, zeros)
    return adj.reshape(B, NPER, NPER, DE), x.reshape(B, NPER, DF)
